# baseline (speedup 1.0000x reference)
import functools

import jax
import jax.numpy as jnp
from jax import lax
from jax.experimental import pallas as pl
from jax.experimental.pallas import tpu as pltpu
from jax.experimental.pallas import tpu_sc as plsc

B = 4
N = 4096
D = 2048
L = 16

NC = 2
NS = 16
NW = NC * NS

C = 128
NBLK = D // C
TASKS_PER_WORKER = (B * NBLK) // NW
R = 256
NCHUNK = N // R
G = C // L


def _argmin_body(x_hbm, out_hbm, buf0, buf1, ostage, sem0, sem1):
    wid = lax.axis_index("s") * NC + lax.axis_index("c")

    bufs = (buf0, buf1)
    sems = (sem0, sem1)

    blk = wid % NBLK
    c0 = blk * C

    for t in range(TASKS_PER_WORKER):
        b = wid // NBLK + 2 * t
        row_base = b * N

        def start(i, k):
            return pltpu.async_copy(
                x_hbm.at[pl.ds(row_base + i * R, R), pl.ds(c0, C)],
                bufs[k], sems[k])

        copies = [None] * NCHUNK
        copies[0] = start(0, 0)

        minvs = [jnp.full((L,), jnp.inf, jnp.float32) for _ in range(G)]
        minis = [jnp.zeros((L,), jnp.int32) for _ in range(G)]

        for i in range(NCHUNK):
            k = i % 2
            if i + 1 < NCHUNK:
                copies[i + 1] = start(i + 1, 1 - k)
            copies[i].wait()
            buf = bufs[k]

            def row_body(r, carry, buf=buf, i=i):
                mvs, mis = carry
                ridx = jnp.full((L,), i * R + r, jnp.int32)
                nmvs = []
                nmis = []
                for g in range(G):
                    xv = buf[r, pl.ds(g * L, L)]
                    m = xv < mvs[g]
                    nmvs.append(jnp.where(m, xv, mvs[g]))
                    nmis.append(jnp.where(m, ridx, mis[g]))
                return tuple(nmvs), tuple(nmis)

            minvs, minis = lax.fori_loop(
                0, R, row_body, (tuple(minvs), tuple(minis)))
            minvs = list(minvs)
            minis = list(minis)

        for g in range(G):
            ostage[pl.ds(g * L, L)] = minis[g]
        pltpu.sync_copy(ostage, out_hbm.at[pl.ds(b * D + c0, C)])


@functools.partial(jax.jit, static_argnames=())
def kernel(x):
    x2 = x.reshape(B * N, D)
    mesh = plsc.VectorSubcoreMesh(core_axis_name="c", subcore_axis_name="s")
    out = pl.kernel(
        _argmin_body,
        out_type=jax.ShapeDtypeStruct((B * D,), jnp.int32),
        mesh=mesh,
        scratch_types=[
            pltpu.VMEM((R, C), jnp.float32),
            pltpu.VMEM((R, C), jnp.float32),
            pltpu.VMEM((C,), jnp.int32),
            pltpu.SemaphoreType.DMA,
            pltpu.SemaphoreType.DMA,
        ],
    )(x2)
    return out.reshape(B, D).astype(jnp.int64)
